# baseline (speedup 1.0000x reference)
import functools

import jax
import jax.numpy as jnp
from jax import lax
from jax.experimental import pallas as pl
from jax.experimental.pallas import tpu as pltpu
from jax.experimental.pallas import tpu_sc as plsc

N = 10000
E = 320000
D = 128

NUM_SC = 2
NUM_TILES = 16
NUM_W = NUM_SC * NUM_TILES
CHUNK = 80
N_CHUNKS = 125
PAD_N = 10240
ROWS_PER_TILE = PAD_N // NUM_TILES


def _sc_scatter_kernel(h_hbm, src_hbm, dst_hbm, agg_hbm,
                       src_v, dst_v, buf0, buf1, agg_sh,
                       gsem0, gsem1, ssem0, ssem1):
    cid = lax.axis_index("c")
    sid = lax.axis_index("s")
    wid = cid * NUM_TILES + sid

    pltpu.async_copy(src_hbm.at[wid], src_v, gsem0)
    pltpu.async_copy(dst_hbm.at[wid], dst_v, gsem1)

    def zbody(i, _):
        buf0[i // 8, pl.ds((i % 8) * 16, 16)] = jnp.zeros((16,), jnp.float32)
        return ()

    lax.fori_loop(0, CHUNK * D // 16, zbody, ())

    def zcopy(k, _):
        pltpu.sync_copy(buf0, agg_sh.at[pl.ds(sid * ROWS_PER_TILE + k * CHUNK, CHUNK)])
        return ()

    lax.fori_loop(0, ROWS_PER_TILE // CHUNK, zcopy, ())
    pltpu.make_async_copy(src_hbm.at[wid], src_v, gsem0).wait()
    pltpu.make_async_copy(dst_hbm.at[wid], dst_v, gsem1).wait()
    plsc.subcore_barrier()

    def sidx(i):
        return src_v.at[pl.ds(pl.multiple_of(i * CHUNK, 8), CHUNK)]

    pltpu.async_copy(h_hbm.at[sidx(0)], buf0, gsem0)
    pltpu.async_copy(h_hbm.at[sidx(1)], buf1, gsem1)

    def body(j, _):
        i0 = 2 * j
        pltpu.make_async_copy(h_hbm.at[sidx(i0)], buf0, gsem0).wait()
        pltpu.async_copy(buf0, agg_sh.at[dst_v.at[i0]], ssem0, add=True)

        pltpu.make_async_copy(h_hbm.at[sidx(i0 + 1)], buf1, gsem1).wait()
        pltpu.async_copy(buf1, agg_sh.at[dst_v.at[i0 + 1]], ssem1, add=True)

        pltpu.make_async_copy(buf0, agg_sh.at[dst_v.at[i0]], ssem0).wait()
        pltpu.async_copy(h_hbm.at[sidx(i0 + 2)], buf0, gsem0)

        pltpu.make_async_copy(buf1, agg_sh.at[dst_v.at[i0 + 1]], ssem1).wait()

        @pl.when(j < N_CHUNKS // 2 - 1)
        def _():
            pltpu.async_copy(h_hbm.at[sidx(i0 + 3)], buf1, gsem1)

        return ()

    lax.fori_loop(0, N_CHUNKS // 2, body, ())

    last = N_CHUNKS - 1
    pltpu.make_async_copy(h_hbm.at[sidx(last)], buf0, gsem0).wait()
    pltpu.sync_copy(buf0, agg_sh.at[dst_v.at[last]], add=True)

    plsc.subcore_barrier()
    pltpu.sync_copy(
        agg_sh.at[pl.ds(sid * ROWS_PER_TILE, ROWS_PER_TILE)],
        agg_hbm.at[cid, pl.ds(sid * ROWS_PER_TILE, ROWS_PER_TILE)],
    )


def _sc_scatter(h, src, dst):
    mesh = plsc.VectorSubcoreMesh(core_axis_name="c", subcore_axis_name="s")
    k = pl.kernel(
        _sc_scatter_kernel,
        mesh=mesh,
        out_type=jax.ShapeDtypeStruct((NUM_SC, PAD_N, D), jnp.float32),
        scratch_types=[
            pltpu.VMEM((N_CHUNKS * CHUNK,), jnp.int32),
            pltpu.VMEM((N_CHUNKS, CHUNK), jnp.int32),
            pltpu.VMEM((CHUNK, D), jnp.float32),
            pltpu.VMEM((CHUNK, D), jnp.float32),
            pltpu.VMEM_SHARED((PAD_N, D), jnp.float32),
            pltpu.SemaphoreType.DMA,
            pltpu.SemaphoreType.DMA,
            pltpu.SemaphoreType.DMA,
            pltpu.SemaphoreType.DMA,
        ],
    )
    return k(h, src, dst)


ROW_BLK = 2000


def _tc_dense_kernel(h_ref, agg_ref, deg_ref, ws_ref, wn_ref, b_ref, out_ref):
    a = agg_ref[0] + agg_ref[1]
    scale = 1.0 / jnp.clip(deg_ref[...], 1.0, None)
    a = a * scale
    acc = jnp.dot(h_ref[...], ws_ref[...], preferred_element_type=jnp.float32)
    acc += jnp.dot(a, wn_ref[...], preferred_element_type=jnp.float32)
    acc += b_ref[...]
    out_ref[...] = jnp.maximum(acc, 0.0)


def _tc_dense(h, agg_parts, deg, W_self, b_self, W_nei):
    grid = (N // ROW_BLK,)
    return pl.pallas_call(
        _tc_dense_kernel,
        grid=grid,
        in_specs=[
            pl.BlockSpec((ROW_BLK, D), lambda i: (i, 0)),
            pl.BlockSpec((NUM_SC, ROW_BLK, D), lambda i: (0, i, 0)),
            pl.BlockSpec((ROW_BLK, 1), lambda i: (i, 0)),
            pl.BlockSpec((D, D), lambda i: (0, 0)),
            pl.BlockSpec((D, D), lambda i: (0, 0)),
            pl.BlockSpec((1, D), lambda i: (0, 0)),
        ],
        out_specs=pl.BlockSpec((ROW_BLK, D), lambda i: (i, 0)),
        out_shape=jax.ShapeDtypeStruct((N, D), jnp.float32),
    )(h, agg_parts, deg.reshape(N, 1), W_self.T, W_nei.T, b_self.reshape(1, D))


@jax.jit
def kernel(h, edge_index, deg, W_self, b_self, W_nei):
    e = edge_index.astype(jnp.int32)
    src = e[0].reshape(NUM_W, N_CHUNKS * CHUNK)
    dst = e[1].reshape(NUM_W, N_CHUNKS, CHUNK)
    agg_parts = _sc_scatter(h, src, dst)
    return _tc_dense(h, agg_parts, deg, W_self, b_self, W_nei)


# speedup vs baseline: 1.2229x; 1.2229x over previous
import functools

import jax
import jax.numpy as jnp
from jax import lax
from jax.experimental import pallas as pl
from jax.experimental.pallas import tpu as pltpu
from jax.experimental.pallas import tpu_sc as plsc

N = 10000
E = 320000
D = 128

NUM_SC = 2
NUM_TILES = 16
NUM_W = NUM_SC * NUM_TILES
CHUNK = 80
N_CHUNKS = 125
PAD_N = 10240
ROWS_PER_TILE = PAD_N // NUM_TILES


def _sc_scatter_kernel(h_hbm, src_hbm, dst_hbm, agg_hbm,
                       src_v, dst_v, buf0, buf1, agg_sh,
                       gsem0, gsem1, ssem0, ssem1):
    cid = lax.axis_index("c")
    sid = lax.axis_index("s")
    wid = cid * NUM_TILES + sid

    pltpu.async_copy(src_hbm.at[wid], src_v, gsem0)
    pltpu.async_copy(dst_hbm.at[wid], dst_v, gsem1)

    def zbody(i, _):
        buf0[i // 8, pl.ds((i % 8) * 16, 16)] = jnp.zeros((16,), jnp.float32)
        return ()

    lax.fori_loop(0, CHUNK * D // 16, zbody, ())

    def zcopy(k, _):
        pltpu.sync_copy(buf0, agg_sh.at[pl.ds(sid * ROWS_PER_TILE + k * CHUNK, CHUNK)])
        return ()

    lax.fori_loop(0, ROWS_PER_TILE // CHUNK, zcopy, ())
    pltpu.make_async_copy(src_hbm.at[wid], src_v, gsem0).wait()
    pltpu.make_async_copy(dst_hbm.at[wid], dst_v, gsem1).wait()
    plsc.subcore_barrier()

    def sidx(i):
        return src_v.at[pl.ds(pl.multiple_of(i * CHUNK, 8), CHUNK)]

    pltpu.async_copy(h_hbm.at[sidx(0)], buf0, gsem0)
    pltpu.async_copy(h_hbm.at[sidx(1)], buf1, gsem1)

    def body(j, _):
        i0 = 2 * j
        pltpu.make_async_copy(h_hbm.at[sidx(i0)], buf0, gsem0).wait()
        pltpu.sync_copy(buf0, agg_sh.at[dst_v.at[i0]], add=True)
        pltpu.async_copy(h_hbm.at[sidx(i0 + 2)], buf0, gsem0)

        pltpu.make_async_copy(h_hbm.at[sidx(i0 + 1)], buf1, gsem1).wait()
        pltpu.sync_copy(buf1, agg_sh.at[dst_v.at[i0 + 1]], add=True)

        @pl.when(j < N_CHUNKS // 2 - 1)
        def _():
            pltpu.async_copy(h_hbm.at[sidx(i0 + 3)], buf1, gsem1)

        return ()

    lax.fori_loop(0, N_CHUNKS // 2, body, ())

    last = N_CHUNKS - 1
    pltpu.make_async_copy(h_hbm.at[sidx(last)], buf0, gsem0).wait()
    pltpu.sync_copy(buf0, agg_sh.at[dst_v.at[last]], add=True)

    plsc.subcore_barrier()
    pltpu.sync_copy(
        agg_sh.at[pl.ds(sid * ROWS_PER_TILE, ROWS_PER_TILE)],
        agg_hbm.at[cid, pl.ds(sid * ROWS_PER_TILE, ROWS_PER_TILE)],
    )


def _sc_scatter(h, src, dst):
    mesh = plsc.VectorSubcoreMesh(core_axis_name="c", subcore_axis_name="s")
    k = pl.kernel(
        _sc_scatter_kernel,
        mesh=mesh,
        out_type=jax.ShapeDtypeStruct((NUM_SC, PAD_N, D), jnp.float32),
        scratch_types=[
            pltpu.VMEM((N_CHUNKS * CHUNK,), jnp.int32),
            pltpu.VMEM((N_CHUNKS, CHUNK), jnp.int32),
            pltpu.VMEM((CHUNK, D), jnp.float32),
            pltpu.VMEM((CHUNK, D), jnp.float32),
            pltpu.VMEM_SHARED((PAD_N, D), jnp.float32),
            pltpu.SemaphoreType.DMA,
            pltpu.SemaphoreType.DMA,
            pltpu.SemaphoreType.DMA,
            pltpu.SemaphoreType.DMA,
        ],
    )
    return k(h, src, dst)


ROW_BLK = 2000


def _tc_dense_kernel(h_ref, agg_ref, deg_ref, ws_ref, wn_ref, b_ref, out_ref):
    a = agg_ref[0] + agg_ref[1]
    scale = 1.0 / jnp.clip(deg_ref[...], 1.0, None)
    a = a * scale
    acc = jnp.dot(h_ref[...], ws_ref[...], preferred_element_type=jnp.float32)
    acc += jnp.dot(a, wn_ref[...], preferred_element_type=jnp.float32)
    acc += b_ref[...]
    out_ref[...] = jnp.maximum(acc, 0.0)


def _tc_dense(h, agg_parts, deg, W_self, b_self, W_nei):
    grid = (N // ROW_BLK,)
    return pl.pallas_call(
        _tc_dense_kernel,
        grid=grid,
        in_specs=[
            pl.BlockSpec((ROW_BLK, D), lambda i: (i, 0)),
            pl.BlockSpec((NUM_SC, ROW_BLK, D), lambda i: (0, i, 0)),
            pl.BlockSpec((ROW_BLK, 1), lambda i: (i, 0)),
            pl.BlockSpec((D, D), lambda i: (0, 0)),
            pl.BlockSpec((D, D), lambda i: (0, 0)),
            pl.BlockSpec((1, D), lambda i: (0, 0)),
        ],
        out_specs=pl.BlockSpec((ROW_BLK, D), lambda i: (i, 0)),
        out_shape=jax.ShapeDtypeStruct((N, D), jnp.float32),
    )(h, agg_parts, deg.reshape(N, 1), W_self.T, W_nei.T, b_self.reshape(1, D))


@jax.jit
def kernel(h, edge_index, deg, W_self, b_self, W_nei):
    e = edge_index.astype(jnp.int32)
    src = e[0].reshape(NUM_W, N_CHUNKS * CHUNK)
    dst = e[1].reshape(NUM_W, N_CHUNKS, CHUNK)
    agg_parts = _sc_scatter(h, src, dst)
    return _tc_dense(h, agg_parts, deg, W_self, b_self, W_nei)
